# baseline (speedup 1.0000x reference)
import dataclasses
import functools

import numpy as np
import jax
import jax.numpy as jnp
from jax import lax
from jax.experimental import pallas as pl
from jax.experimental.pallas import tpu as pltpu
from jax.experimental.pallas import tpu_sc as plsc

_N_LEVELS = 16
_N_FEATS = 2
_LOG2_T = 19
_T = 1 << _LOG2_T
_BASE_RES = 16.0
_PER_LEVEL_SCALE = 1.3819
_HIDDEN = 64
_ENC_DIM = _N_LEVELS * _N_FEATS

_P1 = np.int32(np.int64(2654435761) - (1 << 32))
_P2 = np.int32(805459861)
_MASK = np.int32(_T - 1)

_NW = 32
_C = 128


def _sc_encode(xt, tab2d, scales_rep, n):
    pt = n // _NW
    nch = pt // _C
    ngrp = _C // 16

    mesh = plsc.VectorSubcoreMesh(core_axis_name="c", subcore_axis_name="s")
    cp = pltpu.CompilerParams()
    fields = pltpu.CompilerParams.__dataclass_fields__
    if "needs_layout_passes" in fields:
        cp = dataclasses.replace(cp, needs_layout_passes=False)
    if "use_tc_tiling_on_sc" in fields:
        cp = dataclasses.replace(cp, use_tc_tiling_on_sc=False)

    @functools.partial(
        pl.kernel,
        out_type=jax.ShapeDtypeStruct((_ENC_DIM, n), jnp.float32),
        mesh=mesh,
        compiler_params=cp,
        scratch_types=[
            pltpu.VMEM((3, _C), jnp.float32),
            pltpu.VMEM((16, 16), jnp.float32),
            pltpu.VMEM((8 * _C,), jnp.int32),
            pltpu.VMEM((8, _C), jnp.int32),
            pltpu.VMEM((8, _C), jnp.float32),
            pltpu.VMEM((8 * _C, 16), jnp.float32),
            pltpu.VMEM((_ENC_DIM, _C), jnp.float32),
            pltpu.SemaphoreType.DMA,
        ],
    )
    def enc_kernel(xt_hbm, tab_hbm, scl_hbm, enc_hbm,
                   xbuf, sclv, idxv, lov, wtv, gathv, encb, sem):
        wid = lax.axis_index("s") * 2 + lax.axis_index("c")
        pltpu.sync_copy(scl_hbm, sclv)
        iota = lax.iota(jnp.int32, 16)
        one16 = jnp.full((16,), 1, jnp.int32)
        pbase = wid * pt

        @pl.loop(0, nch)
        def _chunk(ch):
            cbase = pbase + ch * _C
            pltpu.sync_copy(xt_hbm.at[:, pl.ds(cbase, _C)], xbuf)
            for j in range(3):
                for g in range(ngrp):
                    sl = pl.ds(g * 16, 16)
                    xbuf[j, sl] = (xbuf[j, sl] + 1.0) * 0.5

            @pl.loop(0, _N_LEVELS)
            def _lvl(l):
                scale = sclv[l]
                lofs = l * _T

                @pl.loop(0, ngrp)
                def _grp(g):
                    sl = pl.ds(g * 16, 16)
                    xs = xbuf[0, sl]
                    ys = xbuf[1, sl]
                    zs = xbuf[2, sl]
                    px = xs * scale
                    py = ys * scale
                    pz = zs * scale
                    ix = px.astype(jnp.int32)
                    iy = py.astype(jnp.int32)
                    iz = pz.astype(jnp.int32)
                    fx = px - ix.astype(jnp.float32)
                    fy = py - iy.astype(jnp.float32)
                    fz = pz - iz.astype(jnp.float32)
                    hxs = (ix, ix + 1)
                    hy0 = iy * _P1
                    hys = (hy0, hy0 + _P1)
                    hz0 = iz * _P2
                    hzs = (hz0, hz0 + _P2)
                    wxs = (1.0 - fx, fx)
                    wys = (1.0 - fy, fy)
                    wzs = (1.0 - fz, fz)
                    for c in range(8):
                        a = c & 1
                        b = (c >> 1) & 1
                        d = (c >> 2) & 1
                        hh = (hxs[a] ^ hys[b]) ^ hzs[d]
                        fidx = (hh & _MASK) + lofs
                        idxv[pl.ds(c * _C + g * 16, 16)] = fidx >> 3
                        lov[c, sl] = (fidx & 7) * 2
                        wtv[c, sl] = (wxs[a] * wys[b]) * wzs[d]

                pltpu.async_copy(tab_hbm.at[idxv], gathv, sem).wait()

                @pl.loop(0, ngrp)
                def _acc(g):
                    sl = pl.ds(g * 16, 16)
                    ip = g * 16 + iota
                    e0 = jnp.zeros((16,), jnp.float32)
                    e1 = jnp.zeros((16,), jnp.float32)
                    for c in range(8):
                        rows = c * _C + ip
                        col0 = lov[c, sl]
                        f0 = plsc.load_gather(gathv, [rows, col0])
                        f1 = plsc.load_gather(gathv, [rows, col0 + one16])
                        wtc = wtv[c, sl]
                        e0 = e0 + f0 * wtc
                        e1 = e1 + f1 * wtc
                    encb[2 * l, sl] = e0
                    encb[2 * l + 1, sl] = e1

            pltpu.sync_copy(encb, enc_hbm.at[:, pl.ds(cbase, _C)])

    return enc_kernel(xt, tab2d, scales_rep)


def _mlp(enc, w1t, w2t, w3t, n):
    nb = 4096

    def mlp_kernel(e_ref, w1_ref, w2_ref, w3_ref, o_ref):
        e = e_ref[...]
        h1 = jnp.maximum(
            jnp.dot(w1_ref[...], e, preferred_element_type=jnp.float32), 0.0)
        h2 = jnp.maximum(
            jnp.dot(w2_ref[...], h1, preferred_element_type=jnp.float32), 0.0)
        o_ref[...] = jnp.dot(w3_ref[...], h2,
                             preferred_element_type=jnp.float32)

    return pl.pallas_call(
        mlp_kernel,
        grid=(n // nb,),
        in_specs=[
            pl.BlockSpec((_ENC_DIM, nb), lambda i: (0, i)),
            pl.BlockSpec((_HIDDEN, _ENC_DIM), lambda i: (0, 0)),
            pl.BlockSpec((_HIDDEN, _HIDDEN), lambda i: (0, 0)),
            pl.BlockSpec((8, _HIDDEN), lambda i: (0, 0)),
        ],
        out_specs=pl.BlockSpec((8, nb), lambda i: (0, i)),
        out_shape=jax.ShapeDtypeStruct((8, n), jnp.float32),
    )(enc, w1t, w2t, w3t)


def kernel(x, table, W1, W2, W3):
    n = x.shape[0]
    xt = x.T
    tab2d = table.reshape(_N_LEVELS * _T * _N_FEATS // 16, 16)
    scales = np.array(
        [np.float32(_BASE_RES * (_PER_LEVEL_SCALE ** l))
         for l in range(_N_LEVELS)], np.float32)
    scales_rep = jnp.asarray(np.repeat(scales[:, None], 16, axis=1))
    enc = _sc_encode(xt, tab2d, scales_rep, n)
    w3t = jnp.zeros((8, _HIDDEN), jnp.float32).at[0, :].set(W3[:, 0])
    out = _mlp(enc, W1.T, W2.T, w3t, n)
    return out[0].reshape(n, 1)
